# Initial kernel scaffold; baseline (speedup 1.0000x reference)
import jax
import jax.numpy as jnp
from jax.experimental import pallas as pl


def kernel(x, pos, batch, params):
    raise NotImplementedError("write your pallas kernel here")


# speedup vs baseline: 7.6972x; 7.6972x over previous
import functools
import jax
import jax.numpy as jnp
from jax import lax
from jax.experimental import pallas as pl
from jax.experimental.pallas import tpu as pltpu
from jax.experimental.pallas import tpu_sc as plsc

N = 8192
B = 8
K = 10
EPS = 1e-5

TR = 256
NT = N // TR

SC_NC = 2
SC_NS = 16
SC_NW = SC_NC * SC_NS
GCH = 512


def _dot(a, b):
    return jnp.dot(a, b, preferred_element_type=jnp.float32,
                   precision=lax.Precision.DEFAULT)


def _knn_body(f_rows, f_cols, sq_r, sq_c, b_r, b_c, idx_out):
    fr = f_rows[...]
    fc = f_cols[...]
    dot = lax.dot_general(fr, fc, (((1,), (1,)), ((), ())),
                          preferred_element_type=jnp.float32,
                          precision=lax.Precision.DEFAULT)
    d2 = sq_r[:, 0:1] + sq_c[0:1, :] - 2.0 * dot
    pen = jnp.where(b_r[:, 0:1] != b_c[0:1, :], 1e30, 0.0)
    d2 = d2 + pen
    cols = lax.broadcasted_iota(jnp.int32, (TR, N), 1).astype(jnp.float32)
    for k in range(K):
        m = jnp.min(d2, axis=1, keepdims=True)
        c = jnp.min(jnp.where(d2 <= m, cols, 1e30), axis=1, keepdims=True)
        idx_out[:, k:k + 1] = c.astype(jnp.int32)
        d2 = jnp.where(cols == c, 3e38, d2)


def _knn(f, sq_r, sq_c, b_r, b_c):
    return pl.pallas_call(
        _knn_body,
        grid=(NT,),
        in_specs=[
            pl.BlockSpec((TR, 128), lambda i: (i, 0)),
            pl.BlockSpec((N, 128), lambda i: (0, 0)),
            pl.BlockSpec((TR, 8), lambda i: (i, 0)),
            pl.BlockSpec((8, N), lambda i: (0, 0)),
            pl.BlockSpec((TR, 8), lambda i: (i, 0)),
            pl.BlockSpec((8, N), lambda i: (0, 0)),
        ],
        out_specs=pl.BlockSpec((TR, K), lambda i: (i, 0)),
        out_shape=jax.ShapeDtypeStruct((N, K), jnp.int32),
    )(f, f, sq_r, sq_c, b_r, b_c)


def _gather(table, idx_flat):
    nk = idx_flat.shape[0]
    d = table.shape[1]
    per_w = nk // SC_NW
    nch = per_w // GCH
    mesh = plsc.VectorSubcoreMesh(core_axis_name="c", subcore_axis_name="s")

    @functools.partial(
        pl.kernel,
        mesh=mesh,
        out_type=jax.ShapeDtypeStruct((nk, d), jnp.float32),
        scratch_types=[
            pltpu.VMEM((GCH,), jnp.int32),
            pltpu.VMEM((GCH, d), jnp.float32),
            pltpu.SemaphoreType.DMA,
        ],
    )
    def k(idx_hbm, table_hbm, out_hbm, idx_v, rows_v, sem):
        wid = lax.axis_index("s") * SC_NC + lax.axis_index("c")
        base = wid * per_w
        for c in range(nch):
            off = base + c * GCH
            pltpu.sync_copy(idx_hbm.at[pl.ds(off, GCH)], idx_v)
            pltpu.async_copy(table_hbm.at[idx_v], rows_v, sem).wait()
            pltpu.sync_copy(rows_v, out_hbm.at[pl.ds(off, GCH)])

    return k(idx_flat, table)


def _acc_stats(stats_ref, s, ss):
    @pl.when(pl.program_id(0) == 0)
    def _():
        stats_ref[...] = jnp.zeros(stats_ref.shape, stats_ref.dtype)
    stats_ref[0:1, :] += s
    stats_ref[1:2, :] += ss


def _edge_e_body(cf, want_h, xi_ref, g, w, b, out, stats):
    xi = xi_ref[...][:, :cf]
    wv = w[...]
    bv = b[0:1, :]
    s = 0.0
    ss = 0.0
    mx = None
    for k in range(K):
        xj = g[k][:, :cf]
        e = jnp.concatenate([xi, xj - xi], axis=1)
        hk = jnp.maximum(_dot(e, wv) + bv, 0.0)
        if want_h:
            out[k] = hk
        else:
            mx = hk if mx is None else jnp.maximum(mx, hk)
        s = s + jnp.sum(hk, axis=0, keepdims=True)
        ss = ss + jnp.sum(hk * hk, axis=0, keepdims=True)
    if not want_h:
        out[...] = mx
    _acc_stats(stats, s, ss)


def _edge_e(xi_pad, g3, w, b, want_h):
    cf = w.shape[0] // 2
    cout = w.shape[1]
    out_shape = (jax.ShapeDtypeStruct((K, N, cout) if want_h else (N, cout),
                                      jnp.float32),
                 jax.ShapeDtypeStruct((8, cout), jnp.float32))
    main_spec = (pl.BlockSpec((K, TR, cout), lambda i: (0, i, 0)) if want_h
                 else pl.BlockSpec((TR, cout), lambda i: (i, 0)))
    return pl.pallas_call(
        functools.partial(_edge_e_body, cf, want_h),
        grid=(NT,),
        in_specs=[
            pl.BlockSpec((TR, 128), lambda i: (i, 0)),
            pl.BlockSpec((K, TR, 128), lambda i: (0, i, 0)),
            pl.BlockSpec(w.shape, lambda i: (0, 0)),
            pl.BlockSpec((8, cout), lambda i: (0, 0)),
        ],
        out_specs=(main_spec, pl.BlockSpec((8, cout), lambda i: (0, 0))),
        out_shape=out_shape,
    )(xi_pad, g3, w, b)


def _edge_mm_body(want_h, hin, mu, gv, sq, be, w, b, out, stats):
    wv = w[...]
    bv = b[0:1, :]
    muv, gvv, sqv, bev = mu[0:1, :], gv[0:1, :], sq[0:1, :], be[0:1, :]
    s = 0.0
    ss = 0.0
    mx = None
    for k in range(K):
        hn = gvv * (hin[k] - muv) / sqv + bev
        hk = jnp.maximum(_dot(hn, wv) + bv, 0.0)
        if want_h:
            out[k] = hk
        else:
            mx = hk if mx is None else jnp.maximum(mx, hk)
        s = s + jnp.sum(hk, axis=0, keepdims=True)
        ss = ss + jnp.sum(hk * hk, axis=0, keepdims=True)
    if not want_h:
        out[...] = mx
    _acc_stats(stats, s, ss)


def _edge_mm(hin, norm, w, b, want_h):
    cin, cout = w.shape
    out_shape = (jax.ShapeDtypeStruct((K, N, cout) if want_h else (N, cout),
                                      jnp.float32),
                 jax.ShapeDtypeStruct((8, cout), jnp.float32))
    main_spec = (pl.BlockSpec((K, TR, cout), lambda i: (0, i, 0)) if want_h
                 else pl.BlockSpec((TR, cout), lambda i: (i, 0)))
    norm_specs = [pl.BlockSpec((8, cin), lambda i: (0, 0))] * 4
    return pl.pallas_call(
        functools.partial(_edge_mm_body, want_h),
        grid=(NT,),
        in_specs=[pl.BlockSpec((K, TR, cin), lambda i: (0, i, 0))]
        + norm_specs
        + [pl.BlockSpec((cin, cout), lambda i: (0, 0)),
           pl.BlockSpec((8, cout), lambda i: (0, 0))],
        out_specs=(main_spec, pl.BlockSpec((8, cout), lambda i: (0, 0))),
        out_shape=out_shape,
    )(hin, *norm, w, b)


def _l1pool_body(u, bm, w, b, pooled, stats):
    o = jnp.maximum(_dot(u[...], w[...]) + b[0:1, :], 0.0)
    @pl.when(pl.program_id(0) == 0)
    def _():
        pooled[...] = jnp.full(pooled.shape, -3e38, pooled.dtype)
    _acc_stats(stats, jnp.sum(o, axis=0, keepdims=True),
               jnp.sum(o * o, axis=0, keepdims=True))
    for seg in range(B):
        mb = jnp.max(jnp.where(bm[:, seg:seg + 1] > 0.5, o, -3e38),
                     axis=0, keepdims=True)
        pooled[seg:seg + 1, :] = jnp.maximum(pooled[seg:seg + 1, :], mb)


def _l1pool(u, bmask, w, b):
    cin, cout = w.shape
    return pl.pallas_call(
        _l1pool_body,
        grid=(NT,),
        in_specs=[
            pl.BlockSpec((TR, cin), lambda i: (i, 0)),
            pl.BlockSpec((TR, B), lambda i: (i, 0)),
            pl.BlockSpec((cin, cout), lambda i: (0, 0)),
            pl.BlockSpec((8, cout), lambda i: (0, 0)),
        ],
        out_specs=(pl.BlockSpec((B, cout), lambda i: (0, 0)),
                   pl.BlockSpec((8, cout), lambda i: (0, 0))),
        out_shape=(jax.ShapeDtypeStruct((B, cout), jnp.float32),
                   jax.ShapeDtypeStruct((8, cout), jnp.float32)),
    )(u, bmask, w, b)


def _head_body(pn, wm1, bm1, gm1, bem1, wm2, bm2, gm2, bem2, wo, bo, out):
    def lrbn(h, w, bb, g, be):
        h = jnp.maximum(_dot(h, w[...]) + bb[...], 0.0)
        mu = jnp.mean(h, axis=0, keepdims=True)
        var = jnp.mean(h * h, axis=0, keepdims=True) - mu * mu
        return g[...] * (h - mu) * lax.rsqrt(var + EPS) + be[...]

    h = lrbn(pn[...], wm1, bm1, gm1, bem1)
    h = lrbn(h, wm2, bm2, gm2, bem2)
    logits = _dot(h, wo[...]) + bo[...]
    mx = jnp.max(logits, axis=1, keepdims=True)
    z = logits - mx
    out[...] = z - jnp.log(jnp.sum(jnp.exp(z), axis=1, keepdims=True))


def _head(pn, p):
    bm1 = jnp.broadcast_to(p['m1_b'][None, :], (B, 512))
    gm1 = jnp.broadcast_to(p['m1_g'][None, :], (B, 512))
    bem1 = jnp.broadcast_to(p['m1_be'][None, :], (B, 512))
    bm2 = jnp.broadcast_to(p['m2_b'][None, :], (B, 256))
    gm2 = jnp.broadcast_to(p['m2_g'][None, :], (B, 256))
    bem2 = jnp.broadcast_to(p['m2_be'][None, :], (B, 256))
    bo = jnp.broadcast_to(p['out_b'][None, :], (B, 13))
    args = (pn, p['m1_W'], bm1, gm1, bem1,
            p['m2_W'], bm2, gm2, bem2, p['out_W'], bo)
    return pl.pallas_call(
        _head_body,
        out_shape=jax.ShapeDtypeStruct((B, 13), jnp.float32),
    )(*args)


def _bn_apply(h, stats, g, be, count):
    mu = stats[0] / count
    var = stats[1] / count - mu * mu
    return g * (h - mu[None, :]) / jnp.sqrt(var + EPS)[None, :] + be


def _norm8(stats, g, be, count, c):
    mu = stats[0] / count
    var = stats[1] / count - mu * mu
    sq = jnp.sqrt(var + EPS)
    bc = lambda v: jnp.broadcast_to(v[None, :], (8, c))
    return (bc(mu), bc(g), bc(sq), bc(be))


def _knn_inputs(feat, batch_f):
    f = jnp.pad(feat, ((0, 0), (0, 128 - feat.shape[1])))
    sq = jnp.sum(f * f, axis=1)
    return (f,
            jnp.broadcast_to(sq[:, None], (N, 8)),
            jnp.broadcast_to(sq[None, :], (8, N)),
            jnp.broadcast_to(batch_f[:, None], (N, 8)),
            jnp.broadcast_to(batch_f[None, :], (8, N)))


def _bc8(v, c):
    return jnp.broadcast_to(v[None, :], (8, c))


@jax.jit
def kernel(x, pos, batch, params):
    p = params
    batch_f = batch.astype(jnp.float32)
    h0 = jnp.concatenate([x, pos / 255.0], axis=1)

    kin1 = _knn_inputs(h0, batch_f)
    f1p = kin1[0]
    idx1 = _knn(*kin1)
    g1 = _gather(f1p, idx1.T.reshape(-1)).reshape(K, N, 128)
    m1, st1 = _edge_e(f1p, g1, p['c1a_W'], _bc8(p['c1a_b'], 64), want_h=False)
    x1 = _bn_apply(m1, st1, p['c1a_g'], p['c1a_be'], N * K)

    kin2 = _knn_inputs(x1, batch_f)
    f2p = kin2[0]
    idx2 = _knn(*kin2)
    g2 = _gather(f2p, idx2.T.reshape(-1)).reshape(K, N, 128)
    h1, st2a = _edge_e(f2p, g2, p['c2a_W'], _bc8(p['c2a_b'], 64), want_h=True)
    n2a = _norm8(st2a, p['c2a_g'], p['c2a_be'], N * K, 64)
    h2, st2b = _edge_mm(h1, n2a, p['c2b_W'], _bc8(p['c2b_b'], 64), want_h=True)
    n2b = _norm8(st2b, p['c2b_g'], p['c2b_be'], N * K, 64)
    m3, st2c = _edge_mm(h2, n2b, p['c2c_W'], _bc8(p['c2c_b'], 128),
                        want_h=False)
    x2 = _bn_apply(m3, st2c, p['c2c_g'], p['c2c_be'], N * K)

    u = jnp.concatenate([x1, x2], axis=1)
    bmask = (batch[:, None] == jnp.arange(B)[None, :]).astype(jnp.float32)
    pooled_raw, stl = _l1pool(u, bmask, p['l1_W'], _bc8(p['l1_b'], 1024))
    pooled_n = _bn_apply(pooled_raw, stl, p['l1_g'], p['l1_be'], N)

    return _head(pooled_n, p)
